# baseline (speedup 1.0000x reference)
import functools

import jax
import jax.numpy as jnp
from jax import lax
from jax.experimental import pallas as pl
from jax.experimental.pallas import tpu as pltpu
from jax.experimental.pallas import tpu_sc as plsc

N_NODES = 10000
D = 128
N_GRAPHS = 64
N_CLASSES = 10

NC = 2
NS = 16
CH = 128
CPW = 80
NBUF = 2
EPW = 320000 // (NC * NS)
ROWS = 10112
RPW = ROWS // NS

@functools.cache
def _sc_kernels():
    mesh = plsc.VectorSubcoreMesh(core_axis_name="c", subcore_axis_name="s",
                                  num_cores=NC)

    @functools.partial(
        pl.kernel,
        mesh=mesh,
        out_type=jax.ShapeDtypeStruct((NC, ROWS, D), jnp.float32),
        scratch_types=[
            pltpu.VMEM((CPW, CH), jnp.int32),
            pltpu.VMEM((CH, D), jnp.float32),
            pltpu.VMEM_SHARED((ROWS, D), jnp.float32),
            pltpu.SemaphoreType.DMA,
        ],
    )
    def deg_kernel(dstw_hbm, ones_hbm, zer_hbm, out_hbm,
                   dst_v, ones_v, dacc, sem):
        c = lax.axis_index("c")
        s = lax.axis_index("s")
        w = s * NC + c
        pltpu.sync_copy(dstw_hbm.at[w], dst_v)
        pltpu.sync_copy(ones_hbm, ones_v)
        pltpu.sync_copy(zer_hbm, dacc.at[pl.ds(s * RPW, RPW)])
        plsc.subcore_barrier()

        def body(i, carry):
            pltpu.sync_copy(ones_v, dacc.at[dst_v.at[i]], add=True)
            return carry

        lax.fori_loop(0, CPW, body, 0)
        plsc.subcore_barrier()
        pltpu.sync_copy(dacc.at[pl.ds(s * RPW, RPW)],
                        out_hbm.at[c, pl.ds(s * RPW, RPW)])

    @functools.partial(
        pl.kernel,
        mesh=mesh,
        out_type=jax.ShapeDtypeStruct((NC, ROWS, D), jnp.float32),
        scratch_types=[
            pltpu.VMEM((CPW, CH), jnp.int32),
            pltpu.VMEM((CPW, CH), jnp.int32),
            pltpu.VMEM((CH, D), jnp.float32),
            pltpu.VMEM_SHARED((ROWS, D), jnp.float32),
            pltpu.SemaphoreType.DMA,
        ],
    )
    def mp_kernel(y_hbm, srcw_hbm, dstw_hbm, zer_hbm, out_hbm,
                  src_v, dst_v, buf, acc, sem):
        c = lax.axis_index("c")
        s = lax.axis_index("s")
        w = s * NC + c
        pltpu.sync_copy(srcw_hbm.at[w], src_v)
        pltpu.sync_copy(dstw_hbm.at[w], dst_v)
        pltpu.sync_copy(zer_hbm, acc.at[pl.ds(s * RPW, RPW)])
        plsc.subcore_barrier()

        def body(i, carry):
            pltpu.async_copy(y_hbm.at[src_v.at[i]], buf, sem).wait()
            pltpu.sync_copy(buf, acc.at[dst_v.at[i]], add=True)
            return carry

        lax.fori_loop(0, CPW, body, 0)
        plsc.subcore_barrier()
        pltpu.sync_copy(acc.at[pl.ds(s * RPW, RPW)],
                        out_hbm.at[c, pl.ds(s * RPW, RPW)])

    return deg_kernel, mp_kernel


_BN = 2000
_GRID = N_NODES // _BN


def _mm_body(x_ref, w_ref, o_ref):
    o_ref[...] = jnp.dot(x_ref[...], w_ref[...],
                         preferred_element_type=jnp.float32)


def _dinv(da, db):
    return lax.rsqrt(da[..., 0:1] + db[..., 0:1] + 1.0)


def _prep_body(xw_ref, da_ref, db_ref, o_ref):
    o_ref[...] = xw_ref[...] * _dinv(da_ref[...], db_ref[...])


def _layer2_body(p0_ref, p1_ref, y1_ref, da_ref, db_ref, b1_ref, w2_ref, o_ref):
    dinv = _dinv(da_ref[...], db_ref[...])
    h = dinv * (p0_ref[...] + p1_ref[...] + y1_ref[...]) + b1_ref[...]
    h = jnp.maximum(h, 0.0)
    o_ref[...] = dinv * jnp.dot(h, w2_ref[...],
                                preferred_element_type=jnp.float32)


def _final_body(q0_ref, q1_ref, y2_ref, da_ref, db_ref, b2_ref, bat_ref,
                wl_ref, bl_ref, o_ref, sums, counts):
    i = pl.program_id(0)

    @pl.when(i == 0)
    def _():
        sums[...] = jnp.zeros_like(sums)
        counts[...] = jnp.zeros_like(counts)

    dinv = _dinv(da_ref[...], db_ref[...])
    h2 = dinv * (q0_ref[...] + q1_ref[...] + y2_ref[...]) + b2_ref[...]
    b = bat_ref[0, 0, :]
    onehot = (lax.broadcasted_iota(jnp.int32, (N_GRAPHS, _BN), 0)
              == b[None, :]).astype(jnp.float32)
    sums[...] += jnp.dot(onehot, h2, preferred_element_type=jnp.float32)
    counts[...] += jnp.broadcast_to(
        jnp.sum(onehot, axis=1, keepdims=True), (N_GRAPHS, D))

    @pl.when(i == _GRID - 1)
    def _():
        pooled = sums[...] / jnp.maximum(counts[...], 1.0)
        logits = jnp.dot(pooled, wl_ref[...],
                         preferred_element_type=jnp.float32) + bl_ref[...]
        m = jnp.max(logits, axis=1, keepdims=True)
        z = logits - m
        o_ref[...] = z - jnp.log(jnp.sum(jnp.exp(z), axis=1, keepdims=True))


def _row_spec(i):
    return (i, 0)


def _fixed_spec(i):
    return (0, 0)


_mm_call = pl.pallas_call(
    _mm_body,
    grid=(_GRID,),
    in_specs=[pl.BlockSpec((_BN, D), _row_spec),
              pl.BlockSpec((D, D), _fixed_spec)],
    out_specs=pl.BlockSpec((_BN, D), _row_spec),
    out_shape=jax.ShapeDtypeStruct((N_NODES, D), jnp.float32),
)

_prep_call = pl.pallas_call(
    _prep_body,
    grid=(_GRID,),
    in_specs=[pl.BlockSpec((_BN, D), _row_spec),
              pl.BlockSpec((_BN, 16), _row_spec),
              pl.BlockSpec((_BN, 16), _row_spec)],
    out_specs=pl.BlockSpec((_BN, D), _row_spec),
    out_shape=jax.ShapeDtypeStruct((N_NODES, D), jnp.float32),
)

_layer2_call = pl.pallas_call(
    _layer2_body,
    grid=(_GRID,),
    in_specs=[pl.BlockSpec((_BN, D), _row_spec),
              pl.BlockSpec((_BN, D), _row_spec),
              pl.BlockSpec((_BN, D), _row_spec),
              pl.BlockSpec((_BN, 16), _row_spec),
              pl.BlockSpec((_BN, 16), _row_spec),
              pl.BlockSpec((1, D), _fixed_spec),
              pl.BlockSpec((D, D), _fixed_spec)],
    out_specs=pl.BlockSpec((_BN, D), _row_spec),
    out_shape=jax.ShapeDtypeStruct((N_NODES, D), jnp.float32),
)

_final_call = pl.pallas_call(
    _final_body,
    grid=(_GRID,),
    in_specs=[pl.BlockSpec((_BN, D), _row_spec),
              pl.BlockSpec((_BN, D), _row_spec),
              pl.BlockSpec((_BN, D), _row_spec),
              pl.BlockSpec((_BN, 16), _row_spec),
              pl.BlockSpec((_BN, 16), _row_spec),
              pl.BlockSpec((1, D), _fixed_spec),
              pl.BlockSpec((1, 1, _BN), lambda i: (i, 0, 0)),
              pl.BlockSpec((D, N_CLASSES), _fixed_spec),
              pl.BlockSpec((1, N_CLASSES), _fixed_spec)],
    out_specs=pl.BlockSpec((N_GRAPHS, N_CLASSES), _fixed_spec),
    out_shape=jax.ShapeDtypeStruct((N_GRAPHS, N_CLASSES), jnp.float32),
    scratch_shapes=[pltpu.VMEM((N_GRAPHS, D), jnp.float32),
                    pltpu.VMEM((N_GRAPHS, D), jnp.float32)],
)


def kernel(x, edge_index, batch, W1, b1, W2, b2, Wlin, blin):
    nw = NC * NS
    src = edge_index[0].astype(jnp.int32).reshape(nw, EPW)
    dst = edge_index[1].astype(jnp.int32).reshape(nw, EPW)
    padn = CPW * CH - EPW
    pad_src = jnp.zeros((nw, padn), jnp.int32)
    pad_dst = jnp.tile(jnp.arange(N_NODES, N_NODES + 16, dtype=jnp.int32),
                       (nw, padn // 16))
    srcw = jnp.concatenate([src, pad_src], axis=1).reshape(nw, CPW, CH)
    dstw = jnp.concatenate([dst, pad_dst], axis=1).reshape(nw, CPW, CH)

    onesD = jnp.ones((CH, D), jnp.float32)
    zerD = jnp.zeros((RPW, D), jnp.float32)

    deg_kernel, mp_kernel = _sc_kernels()
    degp = deg_kernel(dstw, onesD, zerD)
    dega = degp[0, :N_NODES, :16]
    degb = degp[1, :N_NODES, :16]

    xw1 = _mm_call(x, W1)
    y1 = _prep_call(xw1, dega, degb)

    p = mp_kernel(y1, srcw, dstw, zerD)
    y2 = _layer2_call(p[0, :N_NODES], p[1, :N_NODES], y1, dega, degb,
                      b1.reshape(1, D), W2)

    q = mp_kernel(y2, srcw, dstw, zerD)
    out = _final_call(q[0, :N_NODES], q[1, :N_NODES], y2, dega, degb,
                      b2.reshape(1, D),
                      batch.astype(jnp.int32).reshape(_GRID, 1, _BN),
                      Wlin, blin.reshape(1, N_CLASSES))
    return out


# speedup vs baseline: 2.0951x; 2.0951x over previous
import functools

import jax
import jax.numpy as jnp
from jax import lax
from jax.experimental import pallas as pl
from jax.experimental.pallas import tpu as pltpu
from jax.experimental.pallas import tpu_sc as plsc

N_NODES = 10000
D = 128
N_GRAPHS = 64
N_CLASSES = 10

NC = 2
NS = 16
CH = 128
CPW = 79
NBUF = 2
EPW = 320000 // (NC * NS)
ROWS = 10112
RPW = ROWS // NS

@functools.cache
def _sc_kernels():
    mesh = plsc.VectorSubcoreMesh(core_axis_name="c", subcore_axis_name="s",
                                  num_cores=NC)

    @functools.partial(
        pl.kernel,
        mesh=mesh,
        out_type=jax.ShapeDtypeStruct((NC, ROWS, D), jnp.float32),
        scratch_types=[
            pltpu.VMEM((CPW, CH), jnp.int32),
            pltpu.VMEM((CH, D), jnp.float32),
            pltpu.VMEM_SHARED((ROWS, D), jnp.float32),
            pltpu.SemaphoreType.DMA,
        ],
    )
    def deg_kernel(dstw_hbm, ones_hbm, zer_hbm, out_hbm,
                   dst_v, ones_v, dacc, sem):
        c = lax.axis_index("c")
        s = lax.axis_index("s")
        w = s * NC + c
        pltpu.sync_copy(dstw_hbm.at[w], dst_v)
        pltpu.sync_copy(ones_hbm, ones_v)
        pltpu.sync_copy(zer_hbm, dacc.at[pl.ds(s * RPW, RPW)])
        plsc.subcore_barrier()

        def body(i, carry):
            pltpu.sync_copy(ones_v, dacc.at[dst_v.at[i]], add=True)
            return carry

        lax.fori_loop(0, CPW, body, 0)
        plsc.subcore_barrier()
        pltpu.sync_copy(dacc.at[pl.ds(s * RPW, RPW)],
                        out_hbm.at[c, pl.ds(s * RPW, RPW)])

    @functools.partial(
        pl.kernel,
        mesh=mesh,
        out_type=jax.ShapeDtypeStruct((NC, ROWS, D), jnp.float32),
        scratch_types=[
            pltpu.VMEM((CPW, CH), jnp.int32),
            pltpu.VMEM((CPW, CH), jnp.int32),
            pltpu.VMEM((CH, D), jnp.float32),
            pltpu.VMEM_SHARED((ROWS, D), jnp.float32),
            pltpu.SemaphoreType.DMA,
        ],
    )
    def mp_kernel(y_hbm, srcw_hbm, dstw_hbm, zer_hbm, out_hbm,
                  src_v, dst_v, buf, acc, sem):
        c = lax.axis_index("c")
        s = lax.axis_index("s")
        w = s * NC + c
        pltpu.sync_copy(srcw_hbm.at[w], src_v)
        pltpu.sync_copy(dstw_hbm.at[w], dst_v)
        pltpu.sync_copy(zer_hbm, acc.at[pl.ds(s * RPW, RPW)])
        plsc.subcore_barrier()

        def body(i, carry):
            pltpu.async_copy(y_hbm.at[src_v.at[i]], buf, sem).wait()
            pltpu.sync_copy(buf, acc.at[dst_v.at[i]], add=True)
            return carry

        lax.fori_loop(0, CPW, body, 0)
        plsc.subcore_barrier()
        pltpu.sync_copy(acc.at[pl.ds(s * RPW, RPW)],
                        out_hbm.at[c, pl.ds(s * RPW, RPW)])

    return deg_kernel, mp_kernel


_BN = 2000
_GRID = N_NODES // _BN


def _mm_body(x_ref, w_ref, o_ref):
    o_ref[...] = jnp.dot(x_ref[...], w_ref[...],
                         preferred_element_type=jnp.float32)


def _dinv(da, db):
    return lax.rsqrt(da[..., 0:1] + db[..., 0:1] + 1.0)


def _prep_body(xw_ref, da_ref, db_ref, o_ref):
    o_ref[...] = xw_ref[...] * _dinv(da_ref[...], db_ref[...])


def _layer2_body(p0_ref, p1_ref, y1_ref, da_ref, db_ref, b1_ref, w2_ref, o_ref):
    dinv = _dinv(da_ref[...], db_ref[...])
    h = dinv * (p0_ref[...] + p1_ref[...] + y1_ref[...]) + b1_ref[...]
    h = jnp.maximum(h, 0.0)
    o_ref[...] = dinv * jnp.dot(h, w2_ref[...],
                                preferred_element_type=jnp.float32)


def _final_body(q0_ref, q1_ref, y2_ref, da_ref, db_ref, b2_ref, bat_ref,
                wl_ref, bl_ref, o_ref, sums, counts):
    i = pl.program_id(0)

    @pl.when(i == 0)
    def _():
        sums[...] = jnp.zeros_like(sums)
        counts[...] = jnp.zeros_like(counts)

    dinv = _dinv(da_ref[...], db_ref[...])
    h2 = dinv * (q0_ref[...] + q1_ref[...] + y2_ref[...]) + b2_ref[...]
    b = bat_ref[0, 0, :]
    onehot = (lax.broadcasted_iota(jnp.int32, (N_GRAPHS, _BN), 0)
              == b[None, :]).astype(jnp.float32)
    sums[...] += jnp.dot(onehot, h2, preferred_element_type=jnp.float32)
    counts[...] += jnp.broadcast_to(
        jnp.sum(onehot, axis=1, keepdims=True), (N_GRAPHS, D))

    @pl.when(i == _GRID - 1)
    def _():
        pooled = sums[...] / jnp.maximum(counts[...], 1.0)
        logits = jnp.dot(pooled, wl_ref[...],
                         preferred_element_type=jnp.float32) + bl_ref[...]
        m = jnp.max(logits, axis=1, keepdims=True)
        z = logits - m
        o_ref[...] = z - jnp.log(jnp.sum(jnp.exp(z), axis=1, keepdims=True))


def _row_spec(i):
    return (i, 0)


def _fixed_spec(i):
    return (0, 0)


_mm_call = pl.pallas_call(
    _mm_body,
    grid=(_GRID,),
    in_specs=[pl.BlockSpec((_BN, D), _row_spec),
              pl.BlockSpec((D, D), _fixed_spec)],
    out_specs=pl.BlockSpec((_BN, D), _row_spec),
    out_shape=jax.ShapeDtypeStruct((N_NODES, D), jnp.float32),
)

_prep_call = pl.pallas_call(
    _prep_body,
    grid=(_GRID,),
    in_specs=[pl.BlockSpec((_BN, D), _row_spec),
              pl.BlockSpec((_BN, 16), _row_spec),
              pl.BlockSpec((_BN, 16), _row_spec)],
    out_specs=pl.BlockSpec((_BN, D), _row_spec),
    out_shape=jax.ShapeDtypeStruct((N_NODES, D), jnp.float32),
)

_layer2_call = pl.pallas_call(
    _layer2_body,
    grid=(_GRID,),
    in_specs=[pl.BlockSpec((_BN, D), _row_spec),
              pl.BlockSpec((_BN, D), _row_spec),
              pl.BlockSpec((_BN, D), _row_spec),
              pl.BlockSpec((_BN, 16), _row_spec),
              pl.BlockSpec((_BN, 16), _row_spec),
              pl.BlockSpec((1, D), _fixed_spec),
              pl.BlockSpec((D, D), _fixed_spec)],
    out_specs=pl.BlockSpec((_BN, D), _row_spec),
    out_shape=jax.ShapeDtypeStruct((N_NODES, D), jnp.float32),
)

_final_call = pl.pallas_call(
    _final_body,
    grid=(_GRID,),
    in_specs=[pl.BlockSpec((_BN, D), _row_spec),
              pl.BlockSpec((_BN, D), _row_spec),
              pl.BlockSpec((_BN, D), _row_spec),
              pl.BlockSpec((_BN, 16), _row_spec),
              pl.BlockSpec((_BN, 16), _row_spec),
              pl.BlockSpec((1, D), _fixed_spec),
              pl.BlockSpec((1, 1, _BN), lambda i: (i, 0, 0)),
              pl.BlockSpec((D, N_CLASSES), _fixed_spec),
              pl.BlockSpec((1, N_CLASSES), _fixed_spec)],
    out_specs=pl.BlockSpec((N_GRAPHS, N_CLASSES), _fixed_spec),
    out_shape=jax.ShapeDtypeStruct((N_GRAPHS, N_CLASSES), jnp.float32),
    scratch_shapes=[pltpu.VMEM((N_GRAPHS, D), jnp.float32),
                    pltpu.VMEM((N_GRAPHS, D), jnp.float32)],
)


def kernel(x, edge_index, batch, W1, b1, W2, b2, Wlin, blin):
    nw = NC * NS
    src = edge_index[0].astype(jnp.int32).reshape(nw, EPW)
    dst = edge_index[1].astype(jnp.int32).reshape(nw, EPW)
    padn = CPW * CH - EPW
    npad_rows = ROWS - N_NODES
    woff = jnp.arange(nw, dtype=jnp.int32)[:, None]
    k = jnp.arange(padn, dtype=jnp.int32)[None, :]
    pad_src = (woff * 31 + k * 91) % N_NODES
    pad_dst = N_NODES + (woff * 17 + k) % npad_rows
    srcw = jnp.concatenate([src, pad_src], axis=1).reshape(nw, CPW, CH)
    dstw = jnp.concatenate([dst, pad_dst], axis=1).reshape(nw, CPW, CH)

    onesD = jnp.ones((CH, D), jnp.float32)
    zerD = jnp.zeros((RPW, D), jnp.float32)

    deg_kernel, mp_kernel = _sc_kernels()
    degp = deg_kernel(dstw, onesD, zerD)
    dega = degp[0, :N_NODES, :16]
    degb = degp[1, :N_NODES, :16]

    xw1 = _mm_call(x, W1)
    y1 = _prep_call(xw1, dega, degb)

    p = mp_kernel(y1, srcw, dstw, zerD)
    y2 = _layer2_call(p[0, :N_NODES], p[1, :N_NODES], y1, dega, degb,
                      b1.reshape(1, D), W2)

    q = mp_kernel(y2, srcw, dstw, zerD)
    out = _final_call(q[0, :N_NODES], q[1, :N_NODES], y2, dega, degb,
                      b2.reshape(1, D),
                      batch.astype(jnp.int32).reshape(_GRID, 1, _BN),
                      Wlin, blin.reshape(1, N_CLASSES))
    return out


# speedup vs baseline: 2.3727x; 1.1325x over previous
import functools

import jax
import jax.numpy as jnp
from jax import lax
from jax.experimental import pallas as pl
from jax.experimental.pallas import tpu as pltpu
from jax.experimental.pallas import tpu_sc as plsc

N_NODES = 10000
D = 128
N_GRAPHS = 64
N_CLASSES = 10

NC = 2
NS = 16
CH = 128
CPW = 80
NBUF = 2
EPW = 320000 // (NC * NS)
ROWS = 10112
RPW = ROWS // NS

@functools.cache
def _sc_kernels():
    mesh = plsc.VectorSubcoreMesh(core_axis_name="c", subcore_axis_name="s",
                                  num_cores=NC)

    @functools.partial(
        pl.kernel,
        mesh=mesh,
        out_type=jax.ShapeDtypeStruct((NC, ROWS, D), jnp.float32),
        scratch_types=[
            pltpu.VMEM((CPW, CH), jnp.int32),
            pltpu.VMEM((CH, D), jnp.float32),
            pltpu.VMEM_SHARED((ROWS, D), jnp.float32),
            pltpu.SemaphoreType.DMA,
        ],
    )
    def deg_kernel(dstw_hbm, ones_hbm, zer_hbm, out_hbm,
                   dst_v, ones_v, dacc, sem):
        c = lax.axis_index("c")
        s = lax.axis_index("s")
        w = s * NC + c
        pltpu.sync_copy(dstw_hbm.at[w], dst_v)
        pltpu.sync_copy(ones_hbm, ones_v)
        pltpu.sync_copy(zer_hbm, dacc.at[pl.ds(s * RPW, RPW)])
        plsc.subcore_barrier()

        def body(i, carry):
            pltpu.sync_copy(ones_v, dacc.at[dst_v.at[i]], add=True)
            return carry

        lax.fori_loop(0, CPW, body, 0)
        plsc.subcore_barrier()
        pltpu.sync_copy(dacc.at[pl.ds(s * RPW, RPW)],
                        out_hbm.at[c, pl.ds(s * RPW, RPW)])

    @functools.partial(
        pl.kernel,
        mesh=mesh,
        out_type=jax.ShapeDtypeStruct((NC, ROWS, D), jnp.float32),
        scratch_types=(
            [pltpu.VMEM((NBUF, CH), jnp.int32),
             pltpu.VMEM((CPW, CH), jnp.int32)]
            + [pltpu.VMEM((CH, D), jnp.float32)] * NBUF
            + [pltpu.VMEM_SHARED((ROWS, D), jnp.float32)]
            + [pltpu.SemaphoreType.DMA] * (3 * NBUF)
        ),
    )
    def mp_kernel(y_hbm, srcw_hbm, dstw_hbm, zer_hbm, out_hbm,
                  src_r, dst_v, *rest):
        bufs = rest[:NBUF]
        acc = rest[NBUF]
        isems = rest[NBUF + 1:2 * NBUF + 1]
        gsems = rest[2 * NBUF + 1:3 * NBUF + 1]
        ssems = rest[3 * NBUF + 1:]
        c = lax.axis_index("c")
        s = lax.axis_index("s")
        w = s * NC + c
        pltpu.sync_copy(dstw_hbm.at[w], dst_v)
        pltpu.sync_copy(zer_hbm, acc.at[pl.ds(s * RPW, RPW)])
        plsc.subcore_barrier()

        for b in range(NBUF):
            pltpu.async_copy(srcw_hbm.at[w, b], src_r.at[b], isems[b])
        for b in range(NBUF):
            pltpu.make_async_copy(srcw_hbm.at[w, b], src_r.at[b],
                                  isems[b]).wait()
            pltpu.async_copy(y_hbm.at[src_r.at[b]], bufs[b], gsems[b])

        def body(j, carry):
            base = j * NBUF
            for b in range(NBUF):
                e = base + b
                pltpu.make_async_copy(y_hbm.at[src_r.at[b]], bufs[b],
                                      gsems[b]).wait()

                @pl.when(e + NBUF < CPW)
                def _():
                    pltpu.async_copy(srcw_hbm.at[w, e + NBUF], src_r.at[b],
                                     isems[b])

                pltpu.async_copy(bufs[b], acc.at[dst_v.at[e]], ssems[b],
                                 add=True)
            for b in range(NBUF):
                e = base + b
                pltpu.make_async_copy(bufs[b], acc.at[dst_v.at[e]],
                                      ssems[b]).wait()

                @pl.when(e + NBUF < CPW)
                def _():
                    nxt = e + NBUF
                    pltpu.make_async_copy(srcw_hbm.at[w, nxt], src_r.at[b],
                                          isems[b]).wait()
                    pltpu.async_copy(y_hbm.at[src_r.at[b]], bufs[b],
                                     gsems[b])

            return carry

        lax.fori_loop(0, CPW // NBUF, body, 0)
        plsc.subcore_barrier()
        pltpu.sync_copy(acc.at[pl.ds(s * RPW, RPW)],
                        out_hbm.at[c, pl.ds(s * RPW, RPW)])

    return deg_kernel, mp_kernel


_BN = 2000
_GRID = N_NODES // _BN


def _mm_body(x_ref, w_ref, o_ref):
    o_ref[...] = jnp.dot(x_ref[...], w_ref[...],
                         preferred_element_type=jnp.float32)


def _dinv(da, db):
    return lax.rsqrt(da[..., 0:1] + db[..., 0:1] + 1.0)


def _prep_body(xw_ref, da_ref, db_ref, o_ref):
    o_ref[...] = xw_ref[...] * _dinv(da_ref[...], db_ref[...])


def _layer2_body(p0_ref, p1_ref, y1_ref, da_ref, db_ref, b1_ref, w2_ref, o_ref):
    dinv = _dinv(da_ref[...], db_ref[...])
    h = dinv * (p0_ref[...] + p1_ref[...] + y1_ref[...]) + b1_ref[...]
    h = jnp.maximum(h, 0.0)
    o_ref[...] = dinv * jnp.dot(h, w2_ref[...],
                                preferred_element_type=jnp.float32)


def _final_body(q0_ref, q1_ref, y2_ref, da_ref, db_ref, b2_ref, bat_ref,
                wl_ref, bl_ref, o_ref, sums, counts):
    i = pl.program_id(0)

    @pl.when(i == 0)
    def _():
        sums[...] = jnp.zeros_like(sums)
        counts[...] = jnp.zeros_like(counts)

    dinv = _dinv(da_ref[...], db_ref[...])
    h2 = dinv * (q0_ref[...] + q1_ref[...] + y2_ref[...]) + b2_ref[...]
    b = bat_ref[0, 0, :]
    onehot = (lax.broadcasted_iota(jnp.int32, (N_GRAPHS, _BN), 0)
              == b[None, :]).astype(jnp.float32)
    sums[...] += jnp.dot(onehot, h2, preferred_element_type=jnp.float32)
    counts[...] += jnp.broadcast_to(
        jnp.sum(onehot, axis=1, keepdims=True), (N_GRAPHS, D))

    @pl.when(i == _GRID - 1)
    def _():
        pooled = sums[...] / jnp.maximum(counts[...], 1.0)
        logits = jnp.dot(pooled, wl_ref[...],
                         preferred_element_type=jnp.float32) + bl_ref[...]
        m = jnp.max(logits, axis=1, keepdims=True)
        z = logits - m
        o_ref[...] = z - jnp.log(jnp.sum(jnp.exp(z), axis=1, keepdims=True))


def _row_spec(i):
    return (i, 0)


def _fixed_spec(i):
    return (0, 0)


_mm_call = pl.pallas_call(
    _mm_body,
    grid=(_GRID,),
    in_specs=[pl.BlockSpec((_BN, D), _row_spec),
              pl.BlockSpec((D, D), _fixed_spec)],
    out_specs=pl.BlockSpec((_BN, D), _row_spec),
    out_shape=jax.ShapeDtypeStruct((N_NODES, D), jnp.float32),
)

_prep_call = pl.pallas_call(
    _prep_body,
    grid=(_GRID,),
    in_specs=[pl.BlockSpec((_BN, D), _row_spec),
              pl.BlockSpec((_BN, 16), _row_spec),
              pl.BlockSpec((_BN, 16), _row_spec)],
    out_specs=pl.BlockSpec((_BN, D), _row_spec),
    out_shape=jax.ShapeDtypeStruct((N_NODES, D), jnp.float32),
)

_layer2_call = pl.pallas_call(
    _layer2_body,
    grid=(_GRID,),
    in_specs=[pl.BlockSpec((_BN, D), _row_spec),
              pl.BlockSpec((_BN, D), _row_spec),
              pl.BlockSpec((_BN, D), _row_spec),
              pl.BlockSpec((_BN, 16), _row_spec),
              pl.BlockSpec((_BN, 16), _row_spec),
              pl.BlockSpec((1, D), _fixed_spec),
              pl.BlockSpec((D, D), _fixed_spec)],
    out_specs=pl.BlockSpec((_BN, D), _row_spec),
    out_shape=jax.ShapeDtypeStruct((N_NODES, D), jnp.float32),
)

_final_call = pl.pallas_call(
    _final_body,
    grid=(_GRID,),
    in_specs=[pl.BlockSpec((_BN, D), _row_spec),
              pl.BlockSpec((_BN, D), _row_spec),
              pl.BlockSpec((_BN, D), _row_spec),
              pl.BlockSpec((_BN, 16), _row_spec),
              pl.BlockSpec((_BN, 16), _row_spec),
              pl.BlockSpec((1, D), _fixed_spec),
              pl.BlockSpec((1, 1, _BN), lambda i: (i, 0, 0)),
              pl.BlockSpec((D, N_CLASSES), _fixed_spec),
              pl.BlockSpec((1, N_CLASSES), _fixed_spec)],
    out_specs=pl.BlockSpec((N_GRAPHS, N_CLASSES), _fixed_spec),
    out_shape=jax.ShapeDtypeStruct((N_GRAPHS, N_CLASSES), jnp.float32),
    scratch_shapes=[pltpu.VMEM((N_GRAPHS, D), jnp.float32),
                    pltpu.VMEM((N_GRAPHS, D), jnp.float32)],
)


def kernel(x, edge_index, batch, W1, b1, W2, b2, Wlin, blin):
    nw = NC * NS
    src = edge_index[0].astype(jnp.int32).reshape(nw, EPW)
    dst = edge_index[1].astype(jnp.int32).reshape(nw, EPW)
    padn = CPW * CH - EPW
    npad_rows = ROWS - N_NODES
    woff = jnp.arange(nw, dtype=jnp.int32)[:, None]
    k = jnp.arange(padn, dtype=jnp.int32)[None, :]
    pad_src = (woff * 31 + k * 91) % N_NODES
    pad_dst = N_NODES + (woff * 17 + k) % npad_rows
    srcw = jnp.concatenate([src, pad_src], axis=1).reshape(nw, CPW, CH)
    dstw = jnp.concatenate([dst, pad_dst], axis=1).reshape(nw, CPW, CH)

    onesD = jnp.ones((CH, D), jnp.float32)
    zerD = jnp.zeros((RPW, D), jnp.float32)

    deg_kernel, mp_kernel = _sc_kernels()
    degp = deg_kernel(dstw, onesD, zerD)
    dega = degp[0, :N_NODES, :16]
    degb = degp[1, :N_NODES, :16]

    xw1 = _mm_call(x, W1)
    y1 = _prep_call(xw1, dega, degb)

    p = mp_kernel(y1, srcw, dstw, zerD)
    y2 = _layer2_call(p[0, :N_NODES], p[1, :N_NODES], y1, dega, degb,
                      b1.reshape(1, D), W2)

    q = mp_kernel(y2, srcw, dstw, zerD)
    out = _final_call(q[0, :N_NODES], q[1, :N_NODES], y2, dega, degb,
                      b2.reshape(1, D),
                      batch.astype(jnp.int32).reshape(_GRID, 1, _BN),
                      Wlin, blin.reshape(1, N_CLASSES))
    return out


# speedup vs baseline: 2.3757x; 1.0013x over previous
import functools

import jax
import jax.numpy as jnp
from jax import lax
from jax.experimental import pallas as pl
from jax.experimental.pallas import tpu as pltpu
from jax.experimental.pallas import tpu_sc as plsc

N_NODES = 10000
D = 128
N_GRAPHS = 64
N_CLASSES = 10

NC = 2
NS = 16
CH = 128
CPW = 80
NBUF = 2
EPW = 320000 // (NC * NS)
ROWS = 10112
RPW = ROWS // NS

@functools.cache
def _sc_kernels():
    mesh = plsc.VectorSubcoreMesh(core_axis_name="c", subcore_axis_name="s",
                                  num_cores=NC)

    @functools.partial(
        pl.kernel,
        mesh=mesh,
        out_type=jax.ShapeDtypeStruct((NC, ROWS, D), jnp.float32),
        scratch_types=[
            pltpu.VMEM((CPW, CH), jnp.int32),
            pltpu.VMEM((CH, D), jnp.float32),
            pltpu.VMEM_SHARED((ROWS, D), jnp.float32),
            pltpu.SemaphoreType.DMA,
        ],
    )
    def deg_kernel(dstw_hbm, ones_hbm, zer_hbm, out_hbm,
                   dst_v, ones_v, dacc, sem):
        c = lax.axis_index("c")
        s = lax.axis_index("s")
        w = s * NC + c
        pltpu.sync_copy(dstw_hbm.at[w], dst_v)
        pltpu.sync_copy(ones_hbm, ones_v)
        pltpu.sync_copy(zer_hbm, dacc.at[pl.ds(s * RPW, RPW)])
        plsc.subcore_barrier()

        def body(i, carry):
            pltpu.sync_copy(ones_v, dacc.at[dst_v.at[i]], add=True)
            return carry

        lax.fori_loop(0, CPW, body, 0)
        plsc.subcore_barrier()
        pltpu.sync_copy(dacc.at[pl.ds(s * RPW, RPW)],
                        out_hbm.at[c, pl.ds(s * RPW, RPW)])

    @functools.partial(
        pl.kernel,
        mesh=mesh,
        out_type=jax.ShapeDtypeStruct((NC, ROWS, D), jnp.float32),
        scratch_types=(
            [pltpu.VMEM((NBUF, CH), jnp.int32),
             pltpu.VMEM((CPW, CH), jnp.int32)]
            + [pltpu.VMEM((CH, D), jnp.float32)] * NBUF
            + [pltpu.VMEM_SHARED((ROWS, D), jnp.float32)]
            + [pltpu.SemaphoreType.DMA] * (3 * NBUF)
        ),
    )
    def mp_kernel(y_hbm, srcw_hbm, dstw_hbm, zer_hbm, out_hbm,
                  src_r, dst_v, *rest):
        bufs = rest[:NBUF]
        acc = rest[NBUF]
        isems = rest[NBUF + 1:2 * NBUF + 1]
        gsems = rest[2 * NBUF + 1:3 * NBUF + 1]
        ssems = rest[3 * NBUF + 1:]
        c = lax.axis_index("c")
        s = lax.axis_index("s")
        w = s * NC + c
        pltpu.sync_copy(dstw_hbm.at[w], dst_v)
        pltpu.sync_copy(zer_hbm, acc.at[pl.ds(s * RPW, RPW)])
        plsc.subcore_barrier()

        for b in range(NBUF):
            pltpu.async_copy(srcw_hbm.at[w, b], src_r.at[b], isems[b])
        for b in range(NBUF):
            pltpu.make_async_copy(srcw_hbm.at[w, b], src_r.at[b],
                                  isems[b]).wait()
            pltpu.async_copy(y_hbm.at[src_r.at[b]], bufs[b], gsems[b])

        def body(j, carry):
            base = j * NBUF
            for b in range(NBUF):
                e = base + b
                pltpu.make_async_copy(y_hbm.at[src_r.at[b]], bufs[b],
                                      gsems[b]).wait()

                @pl.when(e + NBUF < CPW)
                def _():
                    pltpu.async_copy(srcw_hbm.at[w, e + NBUF], src_r.at[b],
                                     isems[b])

                pltpu.async_copy(bufs[b], acc.at[dst_v.at[e]], ssems[b],
                                 add=True)
            for b in range(NBUF):
                e = base + b
                pltpu.make_async_copy(bufs[b], acc.at[dst_v.at[e]],
                                      ssems[b]).wait()

                @pl.when(e + NBUF < CPW)
                def _():
                    nxt = e + NBUF
                    pltpu.make_async_copy(srcw_hbm.at[w, nxt], src_r.at[b],
                                          isems[b]).wait()
                    pltpu.async_copy(y_hbm.at[src_r.at[b]], bufs[b],
                                     gsems[b])

            return carry

        lax.fori_loop(0, CPW // NBUF, body, 0)
        plsc.subcore_barrier()
        pltpu.sync_copy(acc.at[pl.ds(s * RPW, RPW)],
                        out_hbm.at[c, pl.ds(s * RPW, RPW)])

    return deg_kernel, mp_kernel


_BN = 2000
_GRID = N_NODES // _BN


def _dinv(da, db):
    return lax.rsqrt(da[..., 0:1] + db[..., 0:1] + 1.0)


def _prep_body(x_ref, w_ref, da_ref, db_ref, o_ref):
    xw = jnp.dot(x_ref[...], w_ref[...], preferred_element_type=jnp.float32)
    o_ref[...] = xw * _dinv(da_ref[...], db_ref[...])


def _layer2_body(p0_ref, p1_ref, y1_ref, da_ref, db_ref, b1_ref, w2_ref, o_ref):
    dinv = _dinv(da_ref[...], db_ref[...])
    h = dinv * (p0_ref[...] + p1_ref[...] + y1_ref[...]) + b1_ref[...]
    h = jnp.maximum(h, 0.0)
    o_ref[...] = dinv * jnp.dot(h, w2_ref[...],
                                preferred_element_type=jnp.float32)


def _final_body(q0_ref, q1_ref, y2_ref, da_ref, db_ref, b2_ref, bat_ref,
                wl_ref, bl_ref, o_ref, sums, counts):
    i = pl.program_id(0)

    @pl.when(i == 0)
    def _():
        sums[...] = jnp.zeros_like(sums)
        counts[...] = jnp.zeros_like(counts)

    dinv = _dinv(da_ref[...], db_ref[...])
    h2 = dinv * (q0_ref[...] + q1_ref[...] + y2_ref[...]) + b2_ref[...]
    b = bat_ref[0, 0, :]
    onehot = (lax.broadcasted_iota(jnp.int32, (N_GRAPHS, _BN), 0)
              == b[None, :]).astype(jnp.float32)
    sums[...] += jnp.dot(onehot, h2, preferred_element_type=jnp.float32)
    counts[...] += jnp.broadcast_to(
        jnp.sum(onehot, axis=1, keepdims=True), (N_GRAPHS, D))

    @pl.when(i == _GRID - 1)
    def _():
        pooled = sums[...] / jnp.maximum(counts[...], 1.0)
        logits = jnp.dot(pooled, wl_ref[...],
                         preferred_element_type=jnp.float32) + bl_ref[...]
        m = jnp.max(logits, axis=1, keepdims=True)
        z = logits - m
        o_ref[...] = z - jnp.log(jnp.sum(jnp.exp(z), axis=1, keepdims=True))


def _row_spec(i):
    return (i, 0)


def _fixed_spec(i):
    return (0, 0)


_prep_call = pl.pallas_call(
    _prep_body,
    grid=(_GRID,),
    in_specs=[pl.BlockSpec((_BN, D), _row_spec),
              pl.BlockSpec((D, D), _fixed_spec),
              pl.BlockSpec((_BN, 16), _row_spec),
              pl.BlockSpec((_BN, 16), _row_spec)],
    out_specs=pl.BlockSpec((_BN, D), _row_spec),
    out_shape=jax.ShapeDtypeStruct((N_NODES, D), jnp.float32),
)

_layer2_call = pl.pallas_call(
    _layer2_body,
    grid=(_GRID,),
    in_specs=[pl.BlockSpec((_BN, D), _row_spec),
              pl.BlockSpec((_BN, D), _row_spec),
              pl.BlockSpec((_BN, D), _row_spec),
              pl.BlockSpec((_BN, 16), _row_spec),
              pl.BlockSpec((_BN, 16), _row_spec),
              pl.BlockSpec((1, D), _fixed_spec),
              pl.BlockSpec((D, D), _fixed_spec)],
    out_specs=pl.BlockSpec((_BN, D), _row_spec),
    out_shape=jax.ShapeDtypeStruct((N_NODES, D), jnp.float32),
)

_final_call = pl.pallas_call(
    _final_body,
    grid=(_GRID,),
    in_specs=[pl.BlockSpec((_BN, D), _row_spec),
              pl.BlockSpec((_BN, D), _row_spec),
              pl.BlockSpec((_BN, D), _row_spec),
              pl.BlockSpec((_BN, 16), _row_spec),
              pl.BlockSpec((_BN, 16), _row_spec),
              pl.BlockSpec((1, D), _fixed_spec),
              pl.BlockSpec((1, 1, _BN), lambda i: (i, 0, 0)),
              pl.BlockSpec((D, N_CLASSES), _fixed_spec),
              pl.BlockSpec((1, N_CLASSES), _fixed_spec)],
    out_specs=pl.BlockSpec((N_GRAPHS, N_CLASSES), _fixed_spec),
    out_shape=jax.ShapeDtypeStruct((N_GRAPHS, N_CLASSES), jnp.float32),
    scratch_shapes=[pltpu.VMEM((N_GRAPHS, D), jnp.float32),
                    pltpu.VMEM((N_GRAPHS, D), jnp.float32)],
)


def kernel(x, edge_index, batch, W1, b1, W2, b2, Wlin, blin):
    nw = NC * NS
    src = edge_index[0].astype(jnp.int32).reshape(nw, EPW)
    dst = edge_index[1].astype(jnp.int32).reshape(nw, EPW)
    padn = CPW * CH - EPW
    npad_rows = ROWS - N_NODES
    woff = jnp.arange(nw, dtype=jnp.int32)[:, None]
    k = jnp.arange(padn, dtype=jnp.int32)[None, :]
    pad_src = (woff * 31 + k * 91) % N_NODES
    pad_dst = N_NODES + (woff * 17 + k) % npad_rows
    srcw = jnp.concatenate([src, pad_src], axis=1).reshape(nw, CPW, CH)
    dstw = jnp.concatenate([dst, pad_dst], axis=1).reshape(nw, CPW, CH)

    onesD = jnp.ones((CH, D), jnp.float32)
    zerD = jnp.zeros((RPW, D), jnp.float32)

    deg_kernel, mp_kernel = _sc_kernels()
    degp = deg_kernel(dstw, onesD, zerD)
    dega = degp[0, :N_NODES, :16]
    degb = degp[1, :N_NODES, :16]

    y1 = _prep_call(x, W1, dega, degb)

    p = mp_kernel(y1, srcw, dstw, zerD)
    y2 = _layer2_call(p[0, :N_NODES], p[1, :N_NODES], y1, dega, degb,
                      b1.reshape(1, D), W2)

    q = mp_kernel(y2, srcw, dstw, zerD)
    out = _final_call(q[0, :N_NODES], q[1, :N_NODES], y2, dega, degb,
                      b2.reshape(1, D),
                      batch.astype(jnp.int32).reshape(_GRID, 1, _BN),
                      Wlin, blin.reshape(1, N_CLASSES))
    return out
